# baseline (speedup 1.0000x reference)
import functools

import jax
import jax.numpy as jnp
from jax.experimental import pallas as pl
from jax.experimental.pallas import tpu as pltpu

NB_OBJECTS = 5
DIM_BODY = 10
DIM_OBJECT = 15
DIM_EDGE = 32
HID = 256
D_PHI_OUT = 64
RHO_HID = 256
D_ACT = 4
TILE_B = 1024


def _mlp_kernel(obs_ref, pw_ref, w1_ref, b1_ref, w2_ref, b2_ref,
                rw1_ref, rb1_ref, mw_ref, mb_ref, lw_ref, lb_ref,
                mean_ref, logstd_ref):
    f32 = jnp.float32
    dot = functools.partial(jnp.dot, preferred_element_type=f32)

    w1_body = w1_ref[:DIM_BODY, :]
    w1_obj = w1_ref[DIM_BODY:DIM_BODY + DIM_OBJECT, :]
    w1_ef = w1_ref[DIM_BODY + DIM_OBJECT:, :]
    w2 = w2_ref[:, :]
    b1 = b1_ref[0, :]
    b2 = b2_ref[0, :]

    for r in range(4):
        obs_r = obs_ref[r]
        t_body = dot(obs_r[:, :DIM_BODY], w1_body) + b1
        agg = jnp.zeros((TILE_B, D_PHI_OUT), dtype=f32)
        for i in range(NB_OBJECTS):
            lo = DIM_BODY + DIM_OBJECT * i
            obj = obs_r[:, lo:lo + DIM_OBJECT]
            ef = pw_ref[i, :, DIM_EDGE * r:DIM_EDGE * (r + 1)]
            h1 = jax.nn.relu(t_body + dot(obj, w1_obj) + dot(ef, w1_ef))
            agg = agg + jax.nn.relu(dot(h1, w2) + b2)
        rr = jax.nn.relu(dot(agg, rw1_ref[:, :]) + rb1_ref[0, :])
        mean_ref[r, :, :] = dot(rr, mw_ref[:, :]) + mb_ref[0, :]
        logstd_ref[r, :, :] = jnp.clip(dot(rr, lw_ref[:, :]) + lb_ref[0, :],
                                       -20.0, 2.0)


def _run_mlp(obs_p3, pw, phi_w1, phi_b1, phi_w2, phi_b2,
             rho_w1, rho_b1, mean_w, mean_b, logstd_w, logstd_b):
    B4 = obs_p3.shape[1]
    njj = B4 // TILE_B

    def rep(shape):
        return pl.BlockSpec(shape, lambda jj: (0,) * len(shape))

    out_shape = (
        jax.ShapeDtypeStruct((4, B4, D_ACT), jnp.float32),
        jax.ShapeDtypeStruct((4, B4, D_ACT), jnp.float32),
    )
    io_spec = pl.BlockSpec((4, TILE_B, D_ACT), lambda jj: (0, jj, 0))
    return pl.pallas_call(
        _mlp_kernel,
        grid=(njj,),
        in_specs=[
            pl.BlockSpec((4, TILE_B, obs_p3.shape[2]), lambda jj: (0, jj, 0)),
            pl.BlockSpec((NB_OBJECTS, TILE_B, 4 * DIM_EDGE),
                         lambda jj: (0, jj, 0)),
            rep(phi_w1.shape),
            rep((1, HID)),
            rep(phi_w2.shape),
            rep((1, D_PHI_OUT)),
            rep(rho_w1.shape),
            rep((1, RHO_HID)),
            rep(mean_w.shape),
            rep((1, D_ACT)),
            rep(logstd_w.shape),
            rep((1, D_ACT)),
        ],
        out_specs=(io_spec, io_spec),
        out_shape=out_shape,
        compiler_params=pltpu.CompilerParams(
            dimension_semantics=("arbitrary",),
        ),
    )(obs_p3, pw,
      phi_w1, phi_b1.reshape(1, HID),
      phi_w2, phi_b2.reshape(1, D_PHI_OUT),
      rho_w1, rho_b1.reshape(1, RHO_HID),
      mean_w, mean_b.reshape(1, D_ACT),
      logstd_w, logstd_b.reshape(1, D_ACT))


def kernel(obs, edge_features, phi_w1, phi_b1, phi_w2, phi_b2,
           rho_w1, rho_b1, mean_w, mean_b, logstd_w, logstd_b):
    B = obs.shape[0]

    ef_r = edge_features.reshape(NB_OBJECTS, 4, B, DIM_EDGE)
    pooled = jnp.max(ef_r, axis=1)
    pw = jnp.concatenate([pooled[:, r::4, :] for r in range(4)], axis=-1)

    obs_p3 = jnp.stack([obs[r::4] for r in range(4)], axis=0)

    mean_p, logstd_p = _run_mlp(obs_p3, pw, phi_w1, phi_b1, phi_w2, phi_b2,
                                rho_w1, rho_b1, mean_w, mean_b,
                                logstd_w, logstd_b)

    def unperm(x):
        return x.transpose(1, 0, 2).reshape(B, D_ACT)

    return (unperm(mean_p), unperm(logstd_p))


# speedup vs baseline: 5.0197x; 5.0197x over previous
import functools

import jax
import jax.numpy as jnp
from jax.experimental import pallas as pl
from jax.experimental.pallas import tpu as pltpu

NB_OBJECTS = 5
DIM_BODY = 10
DIM_OBJECT = 15
DIM_EDGE = 32
HID = 256
D_PHI_OUT = 64
RHO_HID = 256
D_ACT = 4
TILE_B = 4096
PB4 = TILE_B // 4


def _mlp_kernel(obs_ref, pw_ref, w1_ref, b1_ref, w2_ref, b2_ref,
                rw1_ref, rb1_ref, mw_ref, mb_ref, lw_ref, lb_ref,
                mean_ref, logstd_ref):
    f32 = jnp.float32
    dot = functools.partial(jnp.dot, preferred_element_type=f32)

    w1_body = w1_ref[:DIM_BODY, :]
    w1_obj = w1_ref[DIM_BODY:DIM_BODY + DIM_OBJECT, :]
    w1_ef = w1_ref[DIM_BODY + DIM_OBJECT:, :]
    w2 = w2_ref[:, :]
    b1 = b1_ref[0, :]
    b2 = b2_ref[0, :]

    for r in range(4):
        rows = pl.Slice(r, PB4, 4)
        obs_r = obs_ref[rows, :]
        t_body = dot(obs_r[:, :DIM_BODY], w1_body) + b1
        agg = jnp.zeros((PB4, D_PHI_OUT), dtype=f32)
        for i in range(NB_OBJECTS):
            lo = DIM_BODY + DIM_OBJECT * i
            obj = obs_r[:, lo:lo + DIM_OBJECT]
            ef = pw_ref[i, :, DIM_EDGE * r:DIM_EDGE * (r + 1)]
            h1 = jax.nn.relu(t_body + dot(obj, w1_obj) + dot(ef, w1_ef))
            agg = agg + jax.nn.relu(dot(h1, w2) + b2)
        rr = jax.nn.relu(dot(agg, rw1_ref[:, :]) + rb1_ref[0, :])
        mean_ref[rows, :] = dot(rr, mw_ref[:, :]) + mb_ref[0, :]
        logstd_ref[rows, :] = jnp.clip(dot(rr, lw_ref[:, :]) + lb_ref[0, :],
                                       -20.0, 2.0)


def kernel(obs, edge_features, phi_w1, phi_b1, phi_w2, phi_b2,
           rho_w1, rho_b1, mean_w, mean_b, logstd_w, logstd_b):
    B = obs.shape[0]
    grid = (B // TILE_B,)

    ef_r = edge_features.reshape(NB_OBJECTS, 4, B, DIM_EDGE)
    pw = jnp.max(ef_r, axis=1).reshape(NB_OBJECTS, B // 4, 4 * DIM_EDGE)

    def rep(shape):
        return pl.BlockSpec(shape, lambda jj: (0,) * len(shape))

    out_shape = (
        jax.ShapeDtypeStruct((B, D_ACT), jnp.float32),
        jax.ShapeDtypeStruct((B, D_ACT), jnp.float32),
    )
    io_spec = pl.BlockSpec((TILE_B, D_ACT), lambda jj: (jj, 0))
    return pl.pallas_call(
        _mlp_kernel,
        grid=grid,
        in_specs=[
            pl.BlockSpec((TILE_B, obs.shape[1]), lambda jj: (jj, 0)),
            pl.BlockSpec((NB_OBJECTS, PB4, 4 * DIM_EDGE),
                         lambda jj: (0, jj, 0)),
            rep(phi_w1.shape),
            rep((1, HID)),
            rep(phi_w2.shape),
            rep((1, D_PHI_OUT)),
            rep(rho_w1.shape),
            rep((1, RHO_HID)),
            rep(mean_w.shape),
            rep((1, D_ACT)),
            rep(logstd_w.shape),
            rep((1, D_ACT)),
        ],
        out_specs=(io_spec, io_spec),
        out_shape=out_shape,
        compiler_params=pltpu.CompilerParams(
            dimension_semantics=("arbitrary",),
        ),
    )(obs, pw,
      phi_w1, phi_b1.reshape(1, HID),
      phi_w2, phi_b2.reshape(1, D_PHI_OUT),
      rho_w1, rho_b1.reshape(1, RHO_HID),
      mean_w, mean_b.reshape(1, D_ACT),
      logstd_w, logstd_b.reshape(1, D_ACT))


# speedup vs baseline: 5.8656x; 1.1685x over previous
import functools

import jax
import jax.numpy as jnp
from jax.experimental import pallas as pl
from jax.experimental.pallas import tpu as pltpu

NB_OBJECTS = 5
DIM_BODY = 10
DIM_OBJECT = 15
DIM_EDGE = 32
HID = 256
D_PHI_OUT = 64
RHO_HID = 256
D_ACT = 4
TILE_B = 2048


def _mlp_kernel(obs_ref, pool_ref, w1_ref, b1_ref, w2_ref, b2_ref,
                rw1_ref, rb1_ref, mw_ref, mb_ref, lw_ref, lb_ref,
                mean_ref, logstd_ref):
    f32 = jnp.float32
    dot = functools.partial(jnp.dot, preferred_element_type=f32)

    t_body = dot(obs_ref[:, :DIM_BODY], w1_ref[:DIM_BODY, :]) + b1_ref[0, :]
    w1_obj = w1_ref[DIM_BODY:DIM_BODY + DIM_OBJECT, :]
    w1_ef = w1_ref[DIM_BODY + DIM_OBJECT:, :]
    w2 = w2_ref[:, :]
    b2 = b2_ref[0, :]

    agg = jnp.zeros((TILE_B, D_PHI_OUT), dtype=f32)
    for i in range(NB_OBJECTS):
        lo = DIM_BODY + DIM_OBJECT * i
        obj = obs_ref[:, lo:lo + DIM_OBJECT]
        h1 = jax.nn.relu(t_body + dot(obj, w1_obj) + dot(pool_ref[i], w1_ef))
        agg = agg + jax.nn.relu(dot(h1, w2) + b2)

    r = jax.nn.relu(dot(agg, rw1_ref[:, :]) + rb1_ref[0, :])
    mean_ref[:, :] = dot(r, mw_ref[:, :]) + mb_ref[0, :]
    logstd_ref[:, :] = jnp.clip(dot(r, lw_ref[:, :]) + lb_ref[0, :],
                                -20.0, 2.0)


def kernel(obs, edge_features, phi_w1, phi_b1, phi_w2, phi_b2,
           rho_w1, rho_b1, mean_w, mean_b, logstd_w, logstd_b):
    B = obs.shape[0]
    grid = (B // TILE_B,)

    pooled = jnp.max(edge_features.reshape(NB_OBJECTS, 4, B, DIM_EDGE),
                     axis=1)

    def rep(shape):
        return pl.BlockSpec(shape, lambda jj: (0,) * len(shape))

    out_shape = (
        jax.ShapeDtypeStruct((B, D_ACT), jnp.float32),
        jax.ShapeDtypeStruct((B, D_ACT), jnp.float32),
    )
    io_spec = pl.BlockSpec((TILE_B, D_ACT), lambda jj: (jj, 0))
    return pl.pallas_call(
        _mlp_kernel,
        grid=grid,
        in_specs=[
            pl.BlockSpec((TILE_B, obs.shape[1]), lambda jj: (jj, 0)),
            pl.BlockSpec((NB_OBJECTS, TILE_B, DIM_EDGE),
                         lambda jj: (0, jj, 0)),
            rep(phi_w1.shape),
            rep((1, HID)),
            rep(phi_w2.shape),
            rep((1, D_PHI_OUT)),
            rep(rho_w1.shape),
            rep((1, RHO_HID)),
            rep(mean_w.shape),
            rep((1, D_ACT)),
            rep(logstd_w.shape),
            rep((1, D_ACT)),
        ],
        out_specs=(io_spec, io_spec),
        out_shape=out_shape,
        compiler_params=pltpu.CompilerParams(
            dimension_semantics=("arbitrary",),
        ),
    )(obs, pooled,
      phi_w1, phi_b1.reshape(1, HID),
      phi_w2, phi_b2.reshape(1, D_PHI_OUT),
      rho_w1, rho_b1.reshape(1, RHO_HID),
      mean_w, mean_b.reshape(1, D_ACT),
      logstd_w, logstd_b.reshape(1, D_ACT))
